# Initial kernel scaffold; baseline (speedup 1.0000x reference)
import jax
import jax.numpy as jnp
from jax.experimental import pallas as pl


def kernel(x, u3_params, cu3_params, fc_W, fc_b):
    raise NotImplementedError("write your pallas kernel here")


# speedup vs baseline: 4.4638x; 4.4638x over previous
import functools

import jax
import jax.numpy as jnp
import numpy as np
from jax.experimental import pallas as pl
from jax.experimental.pallas import tpu as pltpu

jax.config.update("jax_default_matmul_precision", "highest")

_NSTATES = 65536
_SHOTS = 8192
_NW = 16
_NB = 4

_KS0 = 0
_KS1 = 42
_KS2 = (0x1BD11BDA ^ _KS0 ^ _KS1) & 0xFFFFFFFF
_TINY = float(np.finfo(np.float32).tiny)


def _circuit_body(coef_ref, meta_ref, out_ref):
    ri = jax.lax.broadcasted_iota(jnp.int32, (256, 256), 0)
    ci = jax.lax.broadcasted_iota(jnp.int32, (256, 256), 1)
    re0 = jnp.where((ri == 0) & (ci == 0), 1.0, 0.0).astype(jnp.float32)
    im0 = jnp.zeros((256, 256), jnp.float32)

    def body(g, carry):
        re, im = carry
        u00r = coef_ref[g, 0]
        u00i = coef_ref[g, 1]
        u01r = coef_ref[g, 2]
        u01i = coef_ref[g, 3]
        u10r = coef_ref[g, 4]
        u10i = coef_ref[g, 5]
        u11r = coef_ref[g, 6]
        u11i = coef_ref[g, 7]
        srow = meta_ref[g, 0]
        scol = meta_ref[g, 1]
        crm = meta_ref[g, 2]
        ccm = meta_ref[g, 3]
        uncond = meta_ref[g, 4]
        p_r = (jnp.bitwise_xor(ri, srow) == ci).astype(jnp.float32)
        p_c = (jnp.bitwise_xor(ri, scol) == ci).astype(jnp.float32)
        hp = jax.lax.Precision.HIGHEST
        pre = jnp.dot(p_r, re, preferred_element_type=jnp.float32, precision=hp)
        pim = jnp.dot(p_r, im, preferred_element_type=jnp.float32, precision=hp)
        pre = jnp.dot(pre, p_c, preferred_element_type=jnp.float32, precision=hp)
        pim = jnp.dot(pim, p_c, preferred_element_type=jnp.float32, precision=hp)
        tb = (jnp.bitwise_and(ri, srow) | jnp.bitwise_and(ci, scol)) != 0
        cxr = jnp.where(tb, u11r, u00r)
        cxi = jnp.where(tb, u11i, u00i)
        cpr = jnp.where(tb, u10r, u01r)
        cpi = jnp.where(tb, u10i, u01i)
        nre = cxr * re - cxi * im + cpr * pre - cpi * pim
        nim = cxr * im + cxi * re + cpr * pim + cpi * pre
        app = ((jnp.bitwise_and(ri, crm) | jnp.bitwise_and(ci, ccm)) + uncond) != 0
        re = jnp.where(app, nre, re)
        im = jnp.where(app, nim, im)
        return re, im

    re, im = jax.lax.fori_loop(0, _NB * 2 * _NW, body, (re0, im0))
    amp = jnp.sqrt(re * re + im * im)
    prob = amp * amp
    out_ref[...] = jnp.log(prob + 1e-12)


def _circuit_call(coefs, meta):
    return pl.pallas_call(
        _circuit_body,
        out_shape=jax.ShapeDtypeStruct((256, 256), jnp.float32),
        in_specs=[
            pl.BlockSpec(memory_space=pltpu.SMEM),
            pl.BlockSpec(memory_space=pltpu.SMEM),
        ],
        out_specs=pl.BlockSpec(memory_space=pltpu.VMEM),
    )(coefs, meta)


def _build_gates(u3_params, cu3_params):
    def entries(p):
        th, ph, lam = p[..., 0], p[..., 1], p[..., 2]
        ct = jnp.cos(th / 2.0).astype(jnp.complex64)
        st = jnp.sin(th / 2.0).astype(jnp.complex64)
        e_l = jnp.exp(1j * lam.astype(jnp.complex64))
        e_p = jnp.exp(1j * ph.astype(jnp.complex64))
        u00 = ct
        u01 = -e_l * st
        u10 = e_p * st
        u11 = e_p * e_l * ct
        return jnp.stack(
            [u00.real, u00.imag, u01.real, u01.imag,
             u10.real, u10.imag, u11.real, u11.imag], axis=-1)

    u3e = entries(u3_params)
    cu3e = entries(cu3_params)
    coefs = jnp.concatenate([u3e, cu3e], axis=1).reshape(_NB * 2 * _NW, 8)
    coefs = coefs.astype(jnp.float32)

    meta = np.zeros((_NB * 2 * _NW, 5), np.int32)
    g = 0
    for _ in range(_NB):
        for w in range(_NW):
            if w < 8:
                meta[g] = (1 << (7 - w), 0, 0, 0, 1)
            else:
                meta[g] = (0, 1 << (15 - w), 0, 0, 1)
            g += 1
        for i in range(_NW):
            c, t = i, (i + 1) % _NW
            srow = (1 << (7 - t)) if t < 8 else 0
            scol = 0 if t < 8 else (1 << (15 - t))
            crm = (1 << (7 - c)) if c < 8 else 0
            ccm = 0 if c < 8 else (1 << (15 - c))
            meta[g] = (srow, scol, crm, ccm, 0)
            g += 1
    return coefs, jnp.asarray(meta)


_SROWS = 64


def _tf_round(x0, x1, r):
    x0 = x0 + x1
    x1 = ((x1 << jnp.uint32(r)) | (x1 >> jnp.uint32(32 - r))) ^ x0
    return x0, x1


def _threefry_bits(n):
    ks0 = jnp.uint32(_KS0)
    ks1 = jnp.uint32(_KS1)
    ks2 = jnp.uint32(_KS2)
    x0 = jnp.zeros_like(n)
    x1 = n + ks1
    for r in (13, 15, 26, 6):
        x0, x1 = _tf_round(x0, x1, r)
    x0 = x0 + ks1
    x1 = x1 + ks2 + jnp.uint32(1)
    for r in (17, 29, 16, 24):
        x0, x1 = _tf_round(x0, x1, r)
    x0 = x0 + ks2
    x1 = x1 + ks0 + jnp.uint32(2)
    for r in (13, 15, 26, 6):
        x0, x1 = _tf_round(x0, x1, r)
    x0 = x0 + ks0
    x1 = x1 + ks1 + jnp.uint32(3)
    for r in (17, 29, 16, 24):
        x0, x1 = _tf_round(x0, x1, r)
    x0 = x0 + ks1
    x1 = x1 + ks2 + jnp.uint32(4)
    for r in (13, 15, 26, 6):
        x0, x1 = _tf_round(x0, x1, r)
    x0 = x0 + ks2
    x1 = x1 + ks0 + jnp.uint32(5)
    return x0 ^ x1


def _make_sample_body(srows):
    nsteps = _NSTATES // 128

    def _sample_body(lg_ref, out_ref):
        pid = pl.program_id(0)
        rows = jax.lax.broadcasted_iota(jnp.int32, (srows, 128), 0)
        lanes = jax.lax.broadcasted_iota(jnp.int32, (srows, 128), 1)
        nbase = ((pid * srows + rows) * _NSTATES + lanes).astype(jnp.uint32)
        mx0 = jnp.full((srows, 128), -jnp.inf, jnp.float32)
        ix0 = jnp.zeros((srows, 128), jnp.int32)
        one_minus_tiny = jnp.float32(1.0) - jnp.float32(_TINY)
        tiny = jnp.float32(_TINY)

        def body(jc, carry):
            mx, ix = carry
            n = nbase + (jc * 128).astype(jnp.uint32)
            bits = _threefry_bits(n)
            fb = (bits >> jnp.uint32(9)) | jnp.uint32(0x3F800000)
            f = jax.lax.bitcast_convert_type(fb, jnp.float32) - jnp.float32(1.0)
            u = jnp.maximum(f * one_minus_tiny + tiny, tiny)
            gum = -jnp.log(-jnp.log(u))
            val = gum + lg_ref[pl.ds(jc, 1), :]
            jj = jc * 128 + lanes
            upd = val > mx
            mx = jnp.where(upd, val, mx)
            ix = jnp.where(upd, jj, ix)
            return mx, ix

        mx, ix = jax.lax.fori_loop(0, nsteps, body, (mx0, ix0))
        rowmax = jnp.max(mx, axis=1, keepdims=True)
        cand = jnp.where(mx == rowmax, ix, _NSTATES)
        winner = jnp.min(cand, axis=1, keepdims=True)
        out_ref[...] = jnp.broadcast_to(winner, (srows, 128))

    return _sample_body


def _sample_call(lg):
    grid = _SHOTS // _SROWS
    return pl.pallas_call(
        _make_sample_body(_SROWS),
        grid=(grid,),
        out_shape=jax.ShapeDtypeStruct((_SHOTS, 128), jnp.int32),
        in_specs=[pl.BlockSpec((_NSTATES // 128, 128), lambda i: (0, 0))],
        out_specs=pl.BlockSpec((_SROWS, 128), lambda i: (i, 0)),
        compiler_params=pltpu.CompilerParams(
            dimension_semantics=(pltpu.PARALLEL,)),
    )(lg)


_SCHUNK = 1024


def _fin_body(smp_ref, fcw_ref, fcb_ref, out_ref, acc_ref):
    pid = pl.program_id(0)

    @pl.when(pid == 0)
    def _():
        acc_ref[...] = jnp.zeros((512, 128), jnp.float32)
        out_ref[...] = jnp.zeros((16, 128), jnp.float32)

    blk = smp_ref[...]
    lanes = jax.lax.broadcasted_iota(jnp.int32, (_SCHUNK, 128), 1)
    oh_lo = (jnp.bitwise_and(blk, 127) == lanes).astype(jnp.float32)
    hi = jnp.right_shift(smp_ref[:, 0:1], 7)
    hic = jax.lax.broadcasted_iota(jnp.int32, (_SCHUNK, 512), 1)
    oh_hi = (hi == hic).astype(jnp.float32)
    acc_ref[...] += jax.lax.dot_general(
        oh_hi, oh_lo, (((0,), (0,)), ((), ())),
        preferred_element_type=jnp.float32)

    @pl.when(pid == _SHOTS // _SCHUNK - 1)
    def _():
        counts = acc_ref[...]
        xq = counts * jnp.float32(1.0 / _SHOTS)
        b = jnp.float32(0.8) * jnp.tanh(jnp.float32(0.1 * 2.0 ** (_NW - 1)) * xq)
        xqv = jnp.exp(jnp.float32(0.3) * jnp.log(b))
        mu = jnp.sum(xqv) * jnp.float32(1.0 / _NSTATES)
        xqc = xqv - mu
        for i in range(10):
            y = jnp.sum(fcw_ref[i] * xqc) + fcb_ref[i]
            out_ref[i, :] = jnp.full((128,), y, jnp.float32)


def _fin_call(smp, fc_w3, fc_b):
    nchunks = _SHOTS // _SCHUNK
    return pl.pallas_call(
        _fin_body,
        grid=(nchunks,),
        out_shape=jax.ShapeDtypeStruct((16, 128), jnp.float32),
        in_specs=[
            pl.BlockSpec((_SCHUNK, 128), lambda i: (i, 0)),
            pl.BlockSpec((10, 512, 128), lambda i: (0, 0, 0)),
            pl.BlockSpec(memory_space=pltpu.SMEM),
        ],
        out_specs=pl.BlockSpec((16, 128), lambda i: (0, 0)),
        scratch_shapes=[pltpu.VMEM((512, 128), jnp.float32)],
    )(smp, fc_w3, fc_b)


def kernel(x, u3_params, cu3_params, fc_W, fc_b):
    coefs, meta = _build_gates(u3_params, cu3_params)
    lg = _circuit_call(coefs, meta).reshape(_NSTATES // 128, 128)
    smp = _sample_call(lg)
    out3 = _fin_call(smp, fc_W.reshape(10, 512, 128), fc_b)
    y = out3[:10, 0]
    return jnp.broadcast_to(y[None, :], (x.shape[0], 10))


# speedup vs baseline: 4.9234x; 1.1030x over previous
import functools

import jax
import jax.numpy as jnp
import numpy as np
from jax.experimental import pallas as pl
from jax.experimental.pallas import tpu as pltpu

jax.config.update("jax_default_matmul_precision", "highest")

_NSTATES = 65536
_SHOTS = 8192
_NW = 16
_NB = 4

_KS0 = 0
_KS1 = 42
_KS2 = (0x1BD11BDA ^ _KS0 ^ _KS1) & 0xFFFFFFFF
_TINY = float(np.finfo(np.float32).tiny)


def _circuit_body(coef_ref, meta_ref, out_ref):
    ri = jax.lax.broadcasted_iota(jnp.int32, (256, 256), 0)
    ci = jax.lax.broadcasted_iota(jnp.int32, (256, 256), 1)
    re0 = jnp.where((ri == 0) & (ci == 0), 1.0, 0.0).astype(jnp.float32)
    im0 = jnp.zeros((256, 256), jnp.float32)

    def body(g, carry):
        re, im = carry
        u00r = coef_ref[g, 0]
        u00i = coef_ref[g, 1]
        u01r = coef_ref[g, 2]
        u01i = coef_ref[g, 3]
        u10r = coef_ref[g, 4]
        u10i = coef_ref[g, 5]
        u11r = coef_ref[g, 6]
        u11i = coef_ref[g, 7]
        srow = meta_ref[g, 0]
        scol = meta_ref[g, 1]
        crm = meta_ref[g, 2]
        ccm = meta_ref[g, 3]
        uncond = meta_ref[g, 4]
        p_r = (jnp.bitwise_xor(ri, srow) == ci).astype(jnp.float32)
        p_c = (jnp.bitwise_xor(ri, scol) == ci).astype(jnp.float32)
        hp = jax.lax.Precision.HIGHEST
        pre = jnp.dot(p_r, re, preferred_element_type=jnp.float32, precision=hp)
        pim = jnp.dot(p_r, im, preferred_element_type=jnp.float32, precision=hp)
        pre = jnp.dot(pre, p_c, preferred_element_type=jnp.float32, precision=hp)
        pim = jnp.dot(pim, p_c, preferred_element_type=jnp.float32, precision=hp)
        tb = (jnp.bitwise_and(ri, srow) | jnp.bitwise_and(ci, scol)) != 0
        cxr = jnp.where(tb, u11r, u00r)
        cxi = jnp.where(tb, u11i, u00i)
        cpr = jnp.where(tb, u10r, u01r)
        cpi = jnp.where(tb, u10i, u01i)
        nre = cxr * re - cxi * im + cpr * pre - cpi * pim
        nim = cxr * im + cxi * re + cpr * pim + cpi * pre
        app = ((jnp.bitwise_and(ri, crm) | jnp.bitwise_and(ci, ccm)) + uncond) != 0
        re = jnp.where(app, nre, re)
        im = jnp.where(app, nim, im)
        return re, im

    re, im = jax.lax.fori_loop(0, _NB * 2 * _NW, body, (re0, im0))
    amp = jnp.sqrt(re * re + im * im)
    prob = amp * amp
    out_ref[...] = jnp.log(prob + 1e-12)


def _circuit_call(coefs, meta):
    return pl.pallas_call(
        _circuit_body,
        out_shape=jax.ShapeDtypeStruct((256, 256), jnp.float32),
        in_specs=[
            pl.BlockSpec(memory_space=pltpu.SMEM),
            pl.BlockSpec(memory_space=pltpu.SMEM),
        ],
        out_specs=pl.BlockSpec(memory_space=pltpu.VMEM),
    )(coefs, meta)


def _build_gates(u3_params, cu3_params):
    def entries(p):
        th, ph, lam = p[..., 0], p[..., 1], p[..., 2]
        ct = jnp.cos(th / 2.0).astype(jnp.complex64)
        st = jnp.sin(th / 2.0).astype(jnp.complex64)
        e_l = jnp.exp(1j * lam.astype(jnp.complex64))
        e_p = jnp.exp(1j * ph.astype(jnp.complex64))
        u00 = ct
        u01 = -e_l * st
        u10 = e_p * st
        u11 = e_p * e_l * ct
        return jnp.stack(
            [u00.real, u00.imag, u01.real, u01.imag,
             u10.real, u10.imag, u11.real, u11.imag], axis=-1)

    u3e = entries(u3_params)
    cu3e = entries(cu3_params)
    coefs = jnp.concatenate([u3e, cu3e], axis=1).reshape(_NB * 2 * _NW, 8)
    coefs = coefs.astype(jnp.float32)

    meta = np.zeros((_NB * 2 * _NW, 5), np.int32)
    g = 0
    for _ in range(_NB):
        for w in range(_NW):
            if w < 8:
                meta[g] = (1 << (7 - w), 0, 0, 0, 1)
            else:
                meta[g] = (0, 1 << (15 - w), 0, 0, 1)
            g += 1
        for i in range(_NW):
            c, t = i, (i + 1) % _NW
            srow = (1 << (7 - t)) if t < 8 else 0
            scol = 0 if t < 8 else (1 << (15 - t))
            crm = (1 << (7 - c)) if c < 8 else 0
            ccm = 0 if c < 8 else (1 << (15 - c))
            meta[g] = (srow, scol, crm, ccm, 0)
            g += 1
    return coefs, jnp.asarray(meta)


_SROWS = 64


def _tf_round(x0, x1, r):
    x0 = x0 + x1
    x1 = ((x1 << jnp.uint32(r)) | (x1 >> jnp.uint32(32 - r))) ^ x0
    return x0, x1


def _threefry_bits(n):
    ks0 = jnp.uint32(_KS0)
    ks1 = jnp.uint32(_KS1)
    ks2 = jnp.uint32(_KS2)
    x0 = jnp.zeros_like(n)
    x1 = n + ks1
    for r in (13, 15, 26, 6):
        x0, x1 = _tf_round(x0, x1, r)
    x0 = x0 + ks1
    x1 = x1 + ks2 + jnp.uint32(1)
    for r in (17, 29, 16, 24):
        x0, x1 = _tf_round(x0, x1, r)
    x0 = x0 + ks2
    x1 = x1 + ks0 + jnp.uint32(2)
    for r in (13, 15, 26, 6):
        x0, x1 = _tf_round(x0, x1, r)
    x0 = x0 + ks0
    x1 = x1 + ks1 + jnp.uint32(3)
    for r in (17, 29, 16, 24):
        x0, x1 = _tf_round(x0, x1, r)
    x0 = x0 + ks1
    x1 = x1 + ks2 + jnp.uint32(4)
    for r in (13, 15, 26, 6):
        x0, x1 = _tf_round(x0, x1, r)
    x0 = x0 + ks2
    x1 = x1 + ks0 + jnp.uint32(5)
    return x0 ^ x1


def _make_sample_body(srows, unroll=4):
    nsteps = _NSTATES // (128 * unroll)

    def _sample_body(lg_ref, out_ref):
        pid = pl.program_id(0)
        rows = jax.lax.broadcasted_iota(jnp.int32, (srows, 128), 0)
        lanes = jax.lax.broadcasted_iota(jnp.int32, (srows, 128), 1)
        nbase = ((pid * srows + rows) * _NSTATES + lanes).astype(jnp.uint32)
        mx0 = jnp.full((srows, 128), -jnp.inf, jnp.float32)
        ix0 = jnp.zeros((srows, 128), jnp.int32)
        one_minus_tiny = jnp.float32(1.0) - jnp.float32(_TINY)
        tiny = jnp.float32(_TINY)

        def body(jc, carry):
            mx, ix = carry
            vals = []
            for k in range(unroll):
                j0 = jc * (128 * unroll) + k * 128
                n = nbase + j0.astype(jnp.uint32)
                bits = _threefry_bits(n)
                fb = (bits >> jnp.uint32(9)) | jnp.uint32(0x3F800000)
                f = jax.lax.bitcast_convert_type(fb, jnp.float32) - jnp.float32(1.0)
                u = jnp.maximum(f * one_minus_tiny + tiny, tiny)
                gum = -jnp.log(-jnp.log(u))
                val = gum + lg_ref[pl.ds(jc * unroll + k, 1), :]
                vals.append((val, j0 + lanes))
            while len(vals) > 1:
                nxt = []
                for (va, ja), (vb, jb) in zip(vals[0::2], vals[1::2]):
                    w = vb > va
                    nxt.append((jnp.where(w, vb, va), jnp.where(w, jb, ja)))
                vals = nxt
            val, jj = vals[0]
            upd = val > mx
            mx = jnp.where(upd, val, mx)
            ix = jnp.where(upd, jj, ix)
            return mx, ix

        mx, ix = jax.lax.fori_loop(0, nsteps, body, (mx0, ix0))
        rowmax = jnp.max(mx, axis=1, keepdims=True)
        cand = jnp.where(mx == rowmax, ix, _NSTATES)
        winner = jnp.min(cand, axis=1, keepdims=True)
        out_ref[...] = jnp.broadcast_to(winner, (srows, 128))

    return _sample_body


def _sample_call(lg):
    grid = _SHOTS // _SROWS
    return pl.pallas_call(
        _make_sample_body(_SROWS),
        grid=(grid,),
        out_shape=jax.ShapeDtypeStruct((_SHOTS, 128), jnp.int32),
        in_specs=[pl.BlockSpec((_NSTATES // 128, 128), lambda i: (0, 0))],
        out_specs=pl.BlockSpec((_SROWS, 128), lambda i: (i, 0)),
        compiler_params=pltpu.CompilerParams(
            dimension_semantics=(pltpu.PARALLEL,)),
    )(lg)


_SCHUNK = 1024


def _fin_body(smp_ref, fcw_ref, fcb_ref, out_ref, acc_ref):
    pid = pl.program_id(0)

    @pl.when(pid == 0)
    def _():
        acc_ref[...] = jnp.zeros((512, 128), jnp.float32)
        out_ref[...] = jnp.zeros((16, 128), jnp.float32)

    blk = smp_ref[...]
    lanes = jax.lax.broadcasted_iota(jnp.int32, (_SCHUNK, 128), 1)
    oh_lo = (jnp.bitwise_and(blk, 127) == lanes).astype(jnp.float32)
    hi = jnp.right_shift(smp_ref[:, 0:1], 7)
    hic = jax.lax.broadcasted_iota(jnp.int32, (_SCHUNK, 512), 1)
    oh_hi = (hi == hic).astype(jnp.float32)
    acc_ref[...] += jax.lax.dot_general(
        oh_hi, oh_lo, (((0,), (0,)), ((), ())),
        preferred_element_type=jnp.float32)

    @pl.when(pid == _SHOTS // _SCHUNK - 1)
    def _():
        counts = acc_ref[...]
        xq = counts * jnp.float32(1.0 / _SHOTS)
        b = jnp.float32(0.8) * jnp.tanh(jnp.float32(0.1 * 2.0 ** (_NW - 1)) * xq)
        xqv = jnp.exp(jnp.float32(0.3) * jnp.log(b))
        mu = jnp.sum(xqv) * jnp.float32(1.0 / _NSTATES)
        xqc = xqv - mu
        for i in range(10):
            y = jnp.sum(fcw_ref[i] * xqc) + fcb_ref[i]
            out_ref[i, :] = jnp.full((128,), y, jnp.float32)


def _fin_call(smp, fc_w3, fc_b):
    nchunks = _SHOTS // _SCHUNK
    return pl.pallas_call(
        _fin_body,
        grid=(nchunks,),
        out_shape=jax.ShapeDtypeStruct((16, 128), jnp.float32),
        in_specs=[
            pl.BlockSpec((_SCHUNK, 128), lambda i: (i, 0)),
            pl.BlockSpec((10, 512, 128), lambda i: (0, 0, 0)),
            pl.BlockSpec(memory_space=pltpu.SMEM),
        ],
        out_specs=pl.BlockSpec((16, 128), lambda i: (0, 0)),
        scratch_shapes=[pltpu.VMEM((512, 128), jnp.float32)],
    )(smp, fc_w3, fc_b)


def kernel(x, u3_params, cu3_params, fc_W, fc_b):
    coefs, meta = _build_gates(u3_params, cu3_params)
    lg = _circuit_call(coefs, meta).reshape(_NSTATES // 128, 128)
    smp = _sample_call(lg)
    out3 = _fin_call(smp, fc_W.reshape(10, 512, 128), fc_b)
    y = out3[:10, 0]
    return jnp.broadcast_to(y[None, :], (x.shape[0], 10))


# speedup vs baseline: 4.9971x; 1.0150x over previous
import functools

import jax
import jax.numpy as jnp
import numpy as np
from jax.experimental import pallas as pl
from jax.experimental.pallas import tpu as pltpu

jax.config.update("jax_default_matmul_precision", "highest")

_NSTATES = 65536
_SHOTS = 8192
_NW = 16
_NB = 4

_KS0 = 0
_KS1 = 42
_KS2 = (0x1BD11BDA ^ _KS0 ^ _KS1) & 0xFFFFFFFF
_TINY = float(np.finfo(np.float32).tiny)


def _circuit_body(coef_ref, meta_ref, out_ref):
    ri = jax.lax.broadcasted_iota(jnp.int32, (256, 256), 0)
    ci = jax.lax.broadcasted_iota(jnp.int32, (256, 256), 1)
    re0 = jnp.where((ri == 0) & (ci == 0), 1.0, 0.0).astype(jnp.float32)
    im0 = jnp.zeros((256, 256), jnp.float32)

    def body(g, carry):
        re, im = carry
        u00r = coef_ref[g, 0]
        u00i = coef_ref[g, 1]
        u01r = coef_ref[g, 2]
        u01i = coef_ref[g, 3]
        u10r = coef_ref[g, 4]
        u10i = coef_ref[g, 5]
        u11r = coef_ref[g, 6]
        u11i = coef_ref[g, 7]
        srow = meta_ref[g, 0]
        scol = meta_ref[g, 1]
        crm = meta_ref[g, 2]
        ccm = meta_ref[g, 3]
        uncond = meta_ref[g, 4]
        p_r = (jnp.bitwise_xor(ri, srow) == ci).astype(jnp.float32)
        p_c = (jnp.bitwise_xor(ri, scol) == ci).astype(jnp.float32)
        hp = jax.lax.Precision.HIGHEST
        pre = jnp.dot(p_r, re, preferred_element_type=jnp.float32, precision=hp)
        pim = jnp.dot(p_r, im, preferred_element_type=jnp.float32, precision=hp)
        pre = jnp.dot(pre, p_c, preferred_element_type=jnp.float32, precision=hp)
        pim = jnp.dot(pim, p_c, preferred_element_type=jnp.float32, precision=hp)
        tb = (jnp.bitwise_and(ri, srow) | jnp.bitwise_and(ci, scol)) != 0
        cxr = jnp.where(tb, u11r, u00r)
        cxi = jnp.where(tb, u11i, u00i)
        cpr = jnp.where(tb, u10r, u01r)
        cpi = jnp.where(tb, u10i, u01i)
        nre = cxr * re - cxi * im + cpr * pre - cpi * pim
        nim = cxr * im + cxi * re + cpr * pim + cpi * pre
        app = ((jnp.bitwise_and(ri, crm) | jnp.bitwise_and(ci, ccm)) + uncond) != 0
        re = jnp.where(app, nre, re)
        im = jnp.where(app, nim, im)
        return re, im

    re, im = jax.lax.fori_loop(0, _NB * 2 * _NW, body, (re0, im0))
    amp = jnp.sqrt(re * re + im * im)
    prob = amp * amp
    out_ref[...] = jnp.log(prob + 1e-12)


def _circuit_call(coefs, meta):
    return pl.pallas_call(
        _circuit_body,
        out_shape=jax.ShapeDtypeStruct((256, 256), jnp.float32),
        in_specs=[
            pl.BlockSpec(memory_space=pltpu.SMEM),
            pl.BlockSpec(memory_space=pltpu.SMEM),
        ],
        out_specs=pl.BlockSpec(memory_space=pltpu.VMEM),
    )(coefs, meta)


def _build_gates(u3_params, cu3_params):
    def entries(p):
        th, ph, lam = p[..., 0], p[..., 1], p[..., 2]
        ct = jnp.cos(th / 2.0).astype(jnp.complex64)
        st = jnp.sin(th / 2.0).astype(jnp.complex64)
        e_l = jnp.exp(1j * lam.astype(jnp.complex64))
        e_p = jnp.exp(1j * ph.astype(jnp.complex64))
        u00 = ct
        u01 = -e_l * st
        u10 = e_p * st
        u11 = e_p * e_l * ct
        return jnp.stack(
            [u00.real, u00.imag, u01.real, u01.imag,
             u10.real, u10.imag, u11.real, u11.imag], axis=-1)

    u3e = entries(u3_params)
    cu3e = entries(cu3_params)
    coefs = jnp.concatenate([u3e, cu3e], axis=1).reshape(_NB * 2 * _NW, 8)
    coefs = coefs.astype(jnp.float32)

    meta = np.zeros((_NB * 2 * _NW, 5), np.int32)
    g = 0
    for _ in range(_NB):
        for w in range(_NW):
            if w < 8:
                meta[g] = (1 << (7 - w), 0, 0, 0, 1)
            else:
                meta[g] = (0, 1 << (15 - w), 0, 0, 1)
            g += 1
        for i in range(_NW):
            c, t = i, (i + 1) % _NW
            srow = (1 << (7 - t)) if t < 8 else 0
            scol = 0 if t < 8 else (1 << (15 - t))
            crm = (1 << (7 - c)) if c < 8 else 0
            ccm = 0 if c < 8 else (1 << (15 - c))
            meta[g] = (srow, scol, crm, ccm, 0)
            g += 1
    return coefs, jnp.asarray(meta)


_SROWS = 64


def _tf_round(x0, x1, r):
    x0 = x0 + x1
    x1 = ((x1 << jnp.uint32(r)) | (x1 >> jnp.uint32(32 - r))) ^ x0
    return x0, x1


def _threefry_bits(x1):
    ks0 = jnp.uint32(_KS0)
    ks1 = jnp.uint32(_KS1)
    ks2 = jnp.uint32(_KS2)
    x0 = x1
    x1 = ((x1 << jnp.uint32(13)) | (x1 >> jnp.uint32(19))) ^ x0
    for r in (15, 26, 6):
        x0, x1 = _tf_round(x0, x1, r)
    x0 = x0 + ks1
    x1 = x1 + ks2 + jnp.uint32(1)
    for r in (17, 29, 16, 24):
        x0, x1 = _tf_round(x0, x1, r)
    x0 = x0 + ks2
    x1 = x1 + ks0 + jnp.uint32(2)
    for r in (13, 15, 26, 6):
        x0, x1 = _tf_round(x0, x1, r)
    x0 = x0 + ks0
    x1 = x1 + ks1 + jnp.uint32(3)
    for r in (17, 29, 16, 24):
        x0, x1 = _tf_round(x0, x1, r)
    x0 = x0 + ks1
    x1 = x1 + ks2 + jnp.uint32(4)
    for r in (13, 15, 26, 6):
        x0, x1 = _tf_round(x0, x1, r)
    x0 = x0 + ks2
    x1 = x1 + ks0 + jnp.uint32(5)
    return x0 ^ x1


def _make_sample_body(srows, unroll=4):
    nsteps = _NSTATES // (128 * unroll)

    def _sample_body(lg_ref, out_ref):
        pid = pl.program_id(0)
        rows = jax.lax.broadcasted_iota(jnp.int32, (srows, 128), 0)
        lanes = jax.lax.broadcasted_iota(jnp.int32, (srows, 128), 1)
        nbase = ((pid * srows + rows) * _NSTATES + lanes + _KS1).astype(jnp.uint32)
        mx0 = jnp.full((srows, 128), -jnp.inf, jnp.float32)
        ix0 = jnp.zeros((srows, 128), jnp.int32)
        tiny = jnp.float32(_TINY)

        def body(jc, carry):
            mx, ix = carry
            vals = []
            for k in range(unroll):
                j0 = jc * (128 * unroll) + k * 128
                bits = _threefry_bits(nbase + j0.astype(jnp.uint32))
                fb = (bits >> jnp.uint32(9)) | jnp.uint32(0x3F800000)
                f = jax.lax.bitcast_convert_type(fb, jnp.float32) - jnp.float32(1.0)
                u = jnp.maximum(f, tiny)
                gum = -jnp.log(-jnp.log(u))
                val = gum + lg_ref[pl.ds(jc * unroll + k, 1), :]
                vals.append((val, j0 + lanes))
            while len(vals) > 1:
                nxt = []
                for (va, ja), (vb, jb) in zip(vals[0::2], vals[1::2]):
                    w = vb > va
                    nxt.append((jnp.where(w, vb, va), jnp.where(w, jb, ja)))
                vals = nxt
            val, jj = vals[0]
            upd = val > mx
            mx = jnp.where(upd, val, mx)
            ix = jnp.where(upd, jj, ix)
            return mx, ix

        mx, ix = jax.lax.fori_loop(0, nsteps, body, (mx0, ix0))
        rowmax = jnp.max(mx, axis=1, keepdims=True)
        cand = jnp.where(mx == rowmax, ix, _NSTATES)
        winner = jnp.min(cand, axis=1, keepdims=True)
        out_ref[...] = jnp.broadcast_to(winner, (srows, 128))

    return _sample_body


def _sample_call(lg):
    grid = _SHOTS // _SROWS
    return pl.pallas_call(
        _make_sample_body(_SROWS),
        grid=(grid,),
        out_shape=jax.ShapeDtypeStruct((_SHOTS, 128), jnp.int32),
        in_specs=[pl.BlockSpec((_NSTATES // 128, 128), lambda i: (0, 0))],
        out_specs=pl.BlockSpec((_SROWS, 128), lambda i: (i, 0)),
        compiler_params=pltpu.CompilerParams(
            dimension_semantics=(pltpu.PARALLEL,)),
    )(lg)


_SCHUNK = 1024


def _fin_body(smp_ref, fcw_ref, fcb_ref, out_ref, acc_ref):
    pid = pl.program_id(0)

    @pl.when(pid == 0)
    def _():
        acc_ref[...] = jnp.zeros((512, 128), jnp.float32)
        out_ref[...] = jnp.zeros((16, 128), jnp.float32)

    blk = smp_ref[...]
    lanes = jax.lax.broadcasted_iota(jnp.int32, (_SCHUNK, 128), 1)
    oh_lo = (jnp.bitwise_and(blk, 127) == lanes).astype(jnp.float32)
    hi = jnp.right_shift(smp_ref[:, 0:1], 7)
    hic = jax.lax.broadcasted_iota(jnp.int32, (_SCHUNK, 512), 1)
    oh_hi = (hi == hic).astype(jnp.float32)
    acc_ref[...] += jax.lax.dot_general(
        oh_hi, oh_lo, (((0,), (0,)), ((), ())),
        preferred_element_type=jnp.float32)

    @pl.when(pid == _SHOTS // _SCHUNK - 1)
    def _():
        counts = acc_ref[...]
        xq = counts * jnp.float32(1.0 / _SHOTS)
        b = jnp.float32(0.8) * jnp.tanh(jnp.float32(0.1 * 2.0 ** (_NW - 1)) * xq)
        xqv = jnp.exp(jnp.float32(0.3) * jnp.log(b))
        mu = jnp.sum(xqv) * jnp.float32(1.0 / _NSTATES)
        xqc = xqv - mu
        for i in range(10):
            y = jnp.sum(fcw_ref[i] * xqc) + fcb_ref[i]
            out_ref[i, :] = jnp.full((128,), y, jnp.float32)


def _fin_call(smp, fc_w3, fc_b):
    nchunks = _SHOTS // _SCHUNK
    return pl.pallas_call(
        _fin_body,
        grid=(nchunks,),
        out_shape=jax.ShapeDtypeStruct((16, 128), jnp.float32),
        in_specs=[
            pl.BlockSpec((_SCHUNK, 128), lambda i: (i, 0)),
            pl.BlockSpec((10, 512, 128), lambda i: (0, 0, 0)),
            pl.BlockSpec(memory_space=pltpu.SMEM),
        ],
        out_specs=pl.BlockSpec((16, 128), lambda i: (0, 0)),
        scratch_shapes=[pltpu.VMEM((512, 128), jnp.float32)],
    )(smp, fc_w3, fc_b)


def kernel(x, u3_params, cu3_params, fc_W, fc_b):
    coefs, meta = _build_gates(u3_params, cu3_params)
    lg = _circuit_call(coefs, meta).reshape(_NSTATES // 128, 128)
    smp = _sample_call(lg)
    out3 = _fin_call(smp, fc_W.reshape(10, 512, 128), fc_b)
    y = out3[:10, 0]
    return jnp.broadcast_to(y[None, :], (x.shape[0], 10))
